# baseline (speedup 1.0000x reference)
import functools

import jax
import jax.numpy as jnp
import numpy as np
from jax import lax
from jax.experimental import pallas as pl
from jax.experimental.pallas import tpu as pltpu

NEG = float(jnp.finfo(jnp.float32).min)


def _net_stage(v, idx, d, km, inv):
    n, r = v.shape
    if d >= 8:
        g = n // (2 * d)
        vv = v.reshape(g, 2, d, r)
        ii = idx.reshape(g, 2, d, r)
        av, bv = vv[:, 0], vv[:, 1]
        ai, bi = ii[:, 0], ii[:, 1]
        wv = jnp.maximum(av, bv)
        lv = jnp.minimum(av, bv)
        a_wins = (av > bv) | ((av == bv) & (ai < bi))
        if km == 0:
            sel = ~a_wins if inv else a_wins
            oav, obv = (lv, wv) if inv else (wv, lv)
        else:
            t = lax.broadcasted_iota(jnp.int32, (g, d, r), 1)
            pdm = ((t & km) != 0) if inv else ((t & km) == 0)
            sel = a_wins == pdm
            oav = jnp.where(pdm, wv, lv)
            obv = jnp.where(pdm, lv, wv)
        oai = jnp.where(sel, ai, bi)
        obi = jnp.where(sel, bi, ai)
        v = jnp.concatenate([oav[:, None], obv[:, None]], 1).reshape(n, r)
        idx = jnp.concatenate([oai[:, None], obi[:, None]], 1).reshape(n, r)
    else:
        s = lax.broadcasted_iota(jnp.int32, (n, r), 0)
        left = (s & d) == 0
        pv = jnp.where(left, jnp.roll(v, -d, 0), jnp.roll(v, d, 0))
        pi = jnp.where(left, jnp.roll(idx, -d, 0), jnp.roll(idx, d, 0))
        me_wins = (v > pv) | ((v == pv) & (idx < pi))
        if km == 0:
            keep = (me_wins ^ left) if inv else ~(me_wins ^ left)
        else:
            pdm = ((s & km) != 0) if inv else ((s & km) == 0)
            keep = (me_wins ^ left) ^ pdm
        v = jnp.where(keep, v, pv)
        idx = jnp.where(keep, idx, pi)
    return v, idx


def _br_sort_stages(n, asc):
    nbits = n.bit_length() - 1
    stages = []
    for m in range(1, nbits + 1):
        km = 0 if m == nbits else (1 << (nbits - 1 - m))
        for a in range(m - 1, -1, -1):
            stages.append((1 << (nbits - 1 - a), km, asc))
    return stages


def _sort_dir(v, idx, asc=False):
    for d, km, inv in _br_sort_stages(v.shape[0], asc):
        v, idx = _net_stage(v, idx, d, km, inv)
    return v, idx


def _merge_desc(av, ai, bv, bi):
    n = av.shape[0]
    win = (av > bv) | ((av == bv) & (ai < bi))
    v = jnp.maximum(av, bv)
    idx = jnp.where(win, ai, bi)
    nbits = n.bit_length() - 1
    for a in range(nbits - 1, -1, -1):
        v, idx = _net_stage(v, idx, 1 << (nbits - 1 - a), 0, False)
    return v, idx


def _unpermute(v, idx):
    n, r = v.shape
    nbits = n.bit_length() - 1
    s = lax.broadcasted_iota(jnp.int32, (n, r), 0)
    for a in range(nbits // 2):
        b = nbits - 1 - a
        d = (1 << b) - (1 << a)
        up = ((s & (1 << a)) != 0) & ((s & (1 << b)) == 0)
        dn = ((s & (1 << a)) == 0) & ((s & (1 << b)) != 0)
        v = jnp.where(up, jnp.roll(v, -d, 0), jnp.where(dn, jnp.roll(v, d, 0), v))
        idx = jnp.where(up, jnp.roll(idx, -d, 0),
                        jnp.where(dn, jnp.roll(idx, d, 0), idx))
    return v, idx


def _body(ia_ref, ja_ref, q_ref, k_ref, w_ref, vout_ref, iout_ref,
          topv_ref, topi_ref, *, n1, rb, cb):
    t = pl.program_id(0)
    i = ia_ref[t]
    j = ja_ref[t]

    kk = k_ref[...]
    acc = jnp.zeros((cb, rb), jnp.float32)
    for n in range(n1):
        ln = lax.dot_general(kk, q_ref[n], (((1,), (1,)), ((), ())),
                             preferred_element_type=jnp.float32)
        lp = jnp.maximum(ln, 0.0).astype(jnp.bfloat16).astype(jnp.float32)
        wn = w_ref[n][None, :].astype(jnp.float32)
        acc = acc + lp * wn
    col = j * cb + lax.broadcasted_iota(jnp.int32, (cb, rb), 0)
    row = i * rb + lax.broadcasted_iota(jnp.int32, (cb, rb), 1)
    v = jnp.where(col <= row, acc, NEG)

    @pl.when(j == 0)
    def _():
        sv, si = _sort_dir(v, col, asc=False)
        topv_ref[...] = sv
        topi_ref[...] = si

    @pl.when(j > 0)
    def _():
        sv, si = _sort_dir(v, col, asc=True)
        mv, mi = _merge_desc(topv_ref[...], topi_ref[...], sv, si)
        topv_ref[...] = mv
        topi_ref[...] = mi

    @pl.when((j + 1) * cb >= (i + 1) * rb)
    def _():
        fv, fi = _unpermute(topv_ref[...], topi_ref[...])
        vout_ref[...] = fv.T
        iout_ref[...] = fi.T


def _topk_scores(q_t, k2, w_t, *, rb, cb, k):
    n1, s1, d = q_t.shape
    s2 = k2.shape[0]
    assert cb == k
    ia, ja = [], []
    for i in range(s1 // rb):
        for j in range((i * rb + rb + cb - 1) // cb):
            ia.append(i)
            ja.append(j)
    body = functools.partial(_body, n1=n1, rb=rb, cb=cb)
    grid_spec = pltpu.PrefetchScalarGridSpec(
        num_scalar_prefetch=2,
        grid=(len(ia),),
        in_specs=[
            pl.BlockSpec((n1, rb, d), lambda t, iar, jar: (0, iar[t], 0)),
            pl.BlockSpec((cb, d), lambda t, iar, jar: (jar[t], 0)),
            pl.BlockSpec((n1, rb), lambda t, iar, jar: (0, iar[t])),
        ],
        out_specs=[
            pl.BlockSpec((rb, k), lambda t, iar, jar: (iar[t], 0)),
            pl.BlockSpec((rb, k), lambda t, iar, jar: (iar[t], 0)),
        ],
        scratch_shapes=[
            pltpu.VMEM((k, rb), jnp.float32),
            pltpu.VMEM((k, rb), jnp.int32),
        ],
    )
    vT, iT = pl.pallas_call(
        body,
        grid_spec=grid_spec,
        out_shape=[
            jax.ShapeDtypeStruct((s1, k), jnp.float32),
            jax.ShapeDtypeStruct((s1, k), jnp.int32),
        ],
        compiler_params=pltpu.CompilerParams(
            dimension_semantics=("arbitrary",),
        ),
    )(jnp.asarray(ia, jnp.int32), jnp.asarray(ja, jnp.int32), q_t, k2, w_t)
    return vT, iT


def kernel(query, key, weights, sparse_count):
    b, s1, n1, d = query.shape
    s2 = key.shape[1]
    q_t = jnp.transpose(query[0], (1, 0, 2)).astype(jnp.bfloat16)
    k2 = key[0, :, 0, :].astype(jnp.bfloat16)
    w_t = weights[0].T.astype(jnp.bfloat16)
    vals, idxs = _topk_scores(q_t, k2, w_t, rb=128, cb=512, k=512)
    values = vals[None]
    zero_dep = jnp.asarray(sparse_count, jnp.int32) - jnp.int32(512)
    indices = idxs[None] + zero_dep
    return indices, values


# speedup vs baseline: 1.0895x; 1.0895x over previous
import functools

import jax
import jax.numpy as jnp
import numpy as np
from jax import lax
from jax.experimental import pallas as pl
from jax.experimental.pallas import tpu as pltpu

NEG = float(jnp.finfo(jnp.float32).min)


def _net_stage(v, idx, d, km, inv):
    n, r = v.shape
    if d >= 8:
        g = n // (2 * d)
        vv = v.reshape(g, 2, d, r)
        ii = idx.reshape(g, 2, d, r)
        av, bv = vv[:, 0], vv[:, 1]
        ai, bi = ii[:, 0], ii[:, 1]
        a_wins = (av > bv) | ((av == bv) & (ai < bi))
        if km == 0:
            sel = ~a_wins if inv else a_wins
        else:
            t = lax.broadcasted_iota(jnp.int32, (g, d, r), 1)
            pdm = ((t & km) != 0) if inv else ((t & km) == 0)
            sel = a_wins == pdm
        oav = jnp.where(sel, av, bv)
        obv = jnp.where(sel, bv, av)
        oai = jnp.where(sel, ai, bi)
        obi = jnp.where(sel, bi, ai)
        v = jnp.concatenate([oav[:, None], obv[:, None]], 1).reshape(n, r)
        idx = jnp.concatenate([oai[:, None], obi[:, None]], 1).reshape(n, r)
    else:
        s = lax.broadcasted_iota(jnp.int32, (n, r), 0)
        left = (s & d) == 0
        pv = jnp.where(left, jnp.roll(v, -d, 0), jnp.roll(v, d, 0))
        pi = jnp.where(left, jnp.roll(idx, -d, 0), jnp.roll(idx, d, 0))
        me_wins = (v > pv) | ((v == pv) & (idx < pi))
        if km == 0:
            keep = (me_wins ^ left) if inv else ~(me_wins ^ left)
        else:
            pdm = ((s & km) != 0) if inv else ((s & km) == 0)
            keep = (me_wins ^ left) ^ pdm
        v = jnp.where(keep, v, pv)
        idx = jnp.where(keep, idx, pi)
    return v, idx


def _br_sort_stages(n, asc):
    nbits = n.bit_length() - 1
    stages = []
    for m in range(1, nbits + 1):
        km = 0 if m == nbits else (1 << (nbits - 1 - m))
        for a in range(m - 1, -1, -1):
            stages.append((1 << (nbits - 1 - a), km, asc))
    return stages


def _sort_dir(v, idx, asc=False):
    for d, km, inv in _br_sort_stages(v.shape[0], asc):
        v, idx = _net_stage(v, idx, d, km, inv)
    return v, idx


def _merge_desc(av, ai, bv, bi):
    n = av.shape[0]
    win = (av > bv) | ((av == bv) & (ai < bi))
    v = jnp.where(win, av, bv)
    idx = jnp.where(win, ai, bi)
    nbits = n.bit_length() - 1
    for a in range(nbits - 1, -1, -1):
        v, idx = _net_stage(v, idx, 1 << (nbits - 1 - a), 0, False)
    return v, idx


def _unpermute(v, idx):
    n, r = v.shape
    nbits = n.bit_length() - 1
    s = lax.broadcasted_iota(jnp.int32, (n, r), 0)
    for a in range(nbits // 2):
        b = nbits - 1 - a
        d = (1 << b) - (1 << a)
        up = ((s & (1 << a)) != 0) & ((s & (1 << b)) == 0)
        dn = ((s & (1 << a)) == 0) & ((s & (1 << b)) != 0)
        v = jnp.where(up, jnp.roll(v, -d, 0), jnp.where(dn, jnp.roll(v, d, 0), v))
        idx = jnp.where(up, jnp.roll(idx, -d, 0),
                        jnp.where(dn, jnp.roll(idx, d, 0), idx))
    return v, idx


def _body(ia_ref, ja_ref, q_ref, k_ref, w_ref, vout_ref, iout_ref,
          topv_ref, topi_ref, *, n1, rb, cb):
    t = pl.program_id(0)
    i = ia_ref[t]
    j = ja_ref[t]

    kk = k_ref[...]
    acc = jnp.zeros((cb, rb), jnp.float32)
    for n in range(n1):
        ln = lax.dot_general(kk, q_ref[n], (((1,), (1,)), ((), ())),
                             preferred_element_type=jnp.float32)
        lp = jnp.maximum(ln, 0.0).astype(jnp.bfloat16).astype(jnp.float32)
        wn = w_ref[n][None, :].astype(jnp.float32)
        acc = acc + lp * wn
    col = j * cb + lax.broadcasted_iota(jnp.int32, (cb, rb), 0)
    row = i * rb + lax.broadcasted_iota(jnp.int32, (cb, rb), 1)
    v = jnp.where(col <= row, acc, NEG)

    @pl.when(j == 0)
    def _():
        sv, si = _sort_dir(v, col, asc=False)
        topv_ref[...] = sv
        topi_ref[...] = si

    @pl.when(j > 0)
    def _():
        sv, si = _sort_dir(v, col, asc=True)
        mv, mi = _merge_desc(topv_ref[...], topi_ref[...], sv, si)
        topv_ref[...] = mv
        topi_ref[...] = mi

    @pl.when((j + 1) * cb >= (i + 1) * rb)
    def _():
        fv, fi = _unpermute(topv_ref[...], topi_ref[...])
        vout_ref[...] = fv.T
        iout_ref[...] = fi.T


def _topk_scores(q_t, k2, w_t, *, rb, cb, k):
    n1, s1, d = q_t.shape
    s2 = k2.shape[0]
    assert cb == k
    ia, ja = [], []
    for i in range(s1 // rb):
        for j in range((i * rb + rb + cb - 1) // cb):
            ia.append(i)
            ja.append(j)
    body = functools.partial(_body, n1=n1, rb=rb, cb=cb)
    grid_spec = pltpu.PrefetchScalarGridSpec(
        num_scalar_prefetch=2,
        grid=(len(ia),),
        in_specs=[
            pl.BlockSpec((n1, rb, d), lambda t, iar, jar: (0, iar[t], 0)),
            pl.BlockSpec((cb, d), lambda t, iar, jar: (jar[t], 0)),
            pl.BlockSpec((n1, rb), lambda t, iar, jar: (0, iar[t])),
        ],
        out_specs=[
            pl.BlockSpec((rb, k), lambda t, iar, jar: (iar[t], 0)),
            pl.BlockSpec((rb, k), lambda t, iar, jar: (iar[t], 0)),
        ],
        scratch_shapes=[
            pltpu.VMEM((k, rb), jnp.float32),
            pltpu.VMEM((k, rb), jnp.int32),
        ],
    )
    vT, iT = pl.pallas_call(
        body,
        grid_spec=grid_spec,
        out_shape=[
            jax.ShapeDtypeStruct((s1, k), jnp.float32),
            jax.ShapeDtypeStruct((s1, k), jnp.int32),
        ],
        compiler_params=pltpu.CompilerParams(
            dimension_semantics=("arbitrary",),
        ),
    )(jnp.asarray(ia, jnp.int32), jnp.asarray(ja, jnp.int32), q_t, k2, w_t)
    return vT, iT


def kernel(query, key, weights, sparse_count):
    b, s1, n1, d = query.shape
    s2 = key.shape[1]
    q_t = jnp.transpose(query[0], (1, 0, 2)).astype(jnp.bfloat16)
    k2 = key[0, :, 0, :].astype(jnp.bfloat16)
    w_t = weights[0].T.astype(jnp.bfloat16)
    vals, idxs = _topk_scores(q_t, k2, w_t, rb=128, cb=512, k=512)
    values = vals[None]
    zero_dep = jnp.asarray(sparse_count, jnp.int32) - jnp.int32(512)
    indices = idxs[None] + zero_dep
    return indices, values
